# baseline (speedup 1.0000x reference)
import functools

import jax
import jax.numpy as jnp
from jax import lax
from jax.experimental import pallas as pl
from jax.experimental.pallas import tpu as pltpu
from jax.experimental.pallas import tpu_sc as plsc

D = 128
HALF = 64
HW = 900
N_PIX = 256 * HW
NUM_COLORS = 10
NW = 32
PPW = N_PIX // NW
CHUNK = 480
N_CHUNKS = PPW // CHUNK
GROUPS = CHUNK // 16


def _sc_kernel(x_hbm, idx_hbm, sp_hbm, ch_hbm, out_hbm,
               x_v, idx_v, sp_v, ch_v):
    wid = lax.axis_index("s") * 2 + lax.axis_index("c")
    base = wid * PPW

    pltpu.sync_copy(sp_hbm, sp_v)
    pltpu.sync_copy(ch_hbm, ch_v)

    lane = lax.iota(jnp.int32, 16)

    def chunk_body(c, _):
        pstart = base + c * CHUNK
        pltpu.sync_copy(x_hbm.at[pl.ds(pstart * D, CHUNK * D)], x_v)
        pltpu.sync_copy(idx_hbm.at[pl.ds(pstart, CHUNK)], idx_v)

        def group_body(g, _):
            p16 = g * 16
            pglob = pstart + p16 + lane
            spbase = lax.rem(pglob, HW) * HALF
            cidx = idx_v[pl.ds(p16, 16)]
            chbase = cidx * HALF
            xbase = (p16 + lane) * D
            for jj in range(HALF):
                xi0 = xbase + jj
                v0 = (plsc.load_gather(x_v, [xi0])
                      + plsc.load_gather(sp_v, [spbase + jj]))
                plsc.store_scatter(x_v, [xi0], v0)
                xi1 = xbase + (HALF + jj)
                v1 = (plsc.load_gather(x_v, [xi1])
                      + plsc.load_gather(ch_v, [chbase + jj]))
                plsc.store_scatter(x_v, [xi1], v1)
            return 0

        lax.fori_loop(0, GROUPS, group_body, 0)
        pltpu.sync_copy(x_v, out_hbm.at[pl.ds(pstart * D, CHUNK * D)])
        return 0

    lax.fori_loop(0, N_CHUNKS, chunk_body, 0)


def kernel(x, color_indices, spatial_pe, chromatic_pe):
    Bb, Hh, Ww, d = x.shape
    xf = x.reshape(N_PIX * D)
    idxf = color_indices.astype(jnp.int32).reshape(N_PIX)
    spf = spatial_pe[:Hh, :Ww, :].reshape(HW * HALF)
    chf = chromatic_pe.reshape(NUM_COLORS * HALF)

    mesh = plsc.VectorSubcoreMesh(core_axis_name="c", subcore_axis_name="s")
    run = pl.kernel(
        _sc_kernel,
        jax.ShapeDtypeStruct((N_PIX * D,), jnp.float32),
        mesh=mesh,
        compiler_params=pltpu.CompilerParams(needs_layout_passes=False),
        scratch_types=[
            pltpu.VMEM((CHUNK * D,), jnp.float32),
            pltpu.VMEM((CHUNK,), jnp.int32),
            pltpu.VMEM((HW * HALF,), jnp.float32),
            pltpu.VMEM((NUM_COLORS * HALF,), jnp.float32),
        ],
    )
    out = run(xf, idxf, spf, chf)
    return out.reshape(Bb, Hh, Ww, d)
